# baseline (speedup 1.0000x reference)
import functools
import jax
import jax.numpy as jnp
from jax import lax
from jax.experimental import pallas as pl
from jax.experimental.pallas import tpu as pltpu
from jax.experimental.pallas import tpu_sc as plsc

NUM_TYPES = 119
N_NODES = 100000
OUT_COLS = NUM_TYPES + 1

B = 8192
GRID = -(-N_NODES // B)
NP = GRID * B


def _tc_body_w(w, t_ref, x_ref, y_ref, z_ref, a_ref):
    t = t_ref[0]
    cls = jax.lax.broadcasted_iota(jnp.int32, (OUT_COLS, w), 0)
    one_hot = (cls == t).astype(jnp.float32)
    x = x_ref[0]
    y = y_ref[0]
    z = z_ref[0]
    s = x * x + y * y + z * z
    norm = jnp.sqrt(s)
    d = jnp.maximum(norm, 1e-12)
    sn = s / (d * d)
    a_ref[:] = jnp.where(cls == NUM_TYPES, sn, one_hot)


def _tc_tail_body(w, t_ref, x_ref, y_ref, z_ref, f_in_ref, a_ref):
    del f_in_ref
    _tc_body_w(w, t_ref, x_ref, y_ref, z_ref, a_ref)


W = 512
CPW = 6
SPAN = CPW * W
NC, NS = 2, 16
NW = NC * NS
SC_NODES = NW * SPAN
GW = W // 16
TAIL = N_NODES - SC_NODES
W2 = 2048


def _sc_body(t_hbm, x_hbm, y_hbm, z_hbm, zz_hbm, out_hbm, tb, xb, yb, zb, buf):
    wid = lax.axis_index("s") * NC + lax.axis_index("c")
    lane = lax.iota(jnp.int32, 16)
    ones = jnp.full((16,), 1.0, jnp.float32)
    zeros = jnp.zeros((16,), jnp.float32)
    c119 = jnp.full((16,), NUM_TYPES, jnp.int32)

    span = wid * SPAN
    pltpu.sync_copy(zz_hbm, buf)
    pltpu.sync_copy(t_hbm.at[pl.ds(span, SPAN)], tb)
    pltpu.sync_copy(x_hbm.at[pl.ds(span, SPAN)], xb)
    pltpu.sync_copy(y_hbm.at[pl.ds(span, SPAN)], yb)
    pltpu.sync_copy(z_hbm.at[pl.ds(span, SPAN)], zb)

    def chunk_step(j, carry):
        lb = j * W

        def group_step(g, carry2):
            off = lb + g * 16
            cols = g * 16 + lane
            t = tb[pl.ds(off, 16)]
            x = xb[pl.ds(off, 16)]
            y = yb[pl.ds(off, 16)]
            z = zb[pl.ds(off, 16)]
            s = x * x + y * y + z * z
            sn = s / jnp.maximum(s, 1e-24)
            plsc.store_scatter(buf, [t, cols], ones)
            plsc.store_scatter(buf, [c119, cols], sn)
            return carry2

        lax.fori_loop(0, GW, group_step, 0)
        pltpu.sync_copy(buf, out_hbm.at[:, pl.ds(span + lb, W)])

        def clear_step(g, carry2):
            off = lb + g * 16
            cols = g * 16 + lane
            t = tb[pl.ds(off, 16)]
            plsc.store_scatter(buf, [t, cols], zeros)
            return carry2

        lax.fori_loop(0, GW, clear_step, 0)
        return carry

    lax.fori_loop(0, CPW, chunk_step, 0)


@functools.cache
def _sc_call():
    return pl.kernel(
        _sc_body,
        out_type=jax.ShapeDtypeStruct((OUT_COLS, N_NODES), jnp.float32),
        mesh=plsc.VectorSubcoreMesh(
            core_axis_name="c", subcore_axis_name="s", num_cores=NC, num_subcores=NS
        ),
        scratch_types=[
            pltpu.VMEM((SPAN,), jnp.int32),
            pltpu.VMEM((SPAN,), jnp.float32),
            pltpu.VMEM((SPAN,), jnp.float32),
            pltpu.VMEM((SPAN,), jnp.float32),
            pltpu.VMEM((OUT_COLS, W), jnp.float32),
        ],
        compiler_params=pltpu.CompilerParams(needs_layout_passes=False),
    )


def kernel(atom_type, pos, spin):
    del pos
    t_flat = atom_type.reshape(N_NODES)
    sx = spin[:, 0]
    sy = spin[:, 1]
    sz = spin[:, 2]

    zeros_chunk = jnp.zeros((OUT_COLS, W), jnp.float32)
    feats_t = _sc_call()(t_flat, sx, sy, sz, zeros_chunk)

    pad = (0, NP - N_NODES)
    t3 = jnp.pad(t_flat, pad).reshape(GRID, 1, B)
    x3 = jnp.pad(sx, pad).reshape(GRID, 1, B)
    y3 = jnp.pad(sy, pad).reshape(GRID, 1, B)
    z3 = jnp.pad(sz, pad).reshape(GRID, 1, B)
    in_spec = pl.BlockSpec((1, 1, B), lambda i: (i, 0, 0))
    out_spec = pl.BlockSpec((OUT_COLS, B), lambda i: (0, i))
    attrs_t = pl.pallas_call(
        functools.partial(_tc_body_w, B),
        grid=(GRID,),
        in_specs=[in_spec, in_spec, in_spec, in_spec],
        out_specs=out_spec,
        out_shape=jax.ShapeDtypeStruct((OUT_COLS, N_NODES), jnp.float32),
    )(t3, x3, y3, z3)

    tpad = (0, W2 - TAIL)
    t_t3 = jnp.pad(t_flat[SC_NODES:], tpad).reshape(1, 1, W2)
    x_t3 = jnp.pad(sx[SC_NODES:], tpad).reshape(1, 1, W2)
    y_t3 = jnp.pad(sy[SC_NODES:], tpad).reshape(1, 1, W2)
    z_t3 = jnp.pad(sz[SC_NODES:], tpad).reshape(1, 1, W2)
    tail_in = pl.BlockSpec((1, 1, W2), lambda i: (0, 0, 0))
    tail_blk = SC_NODES // W2
    feats_t = pl.pallas_call(
        functools.partial(_tc_tail_body, W2),
        grid=(1,),
        in_specs=[tail_in, tail_in, tail_in, tail_in,
                  pl.BlockSpec((OUT_COLS, W2), lambda i: (0, tail_blk))],
        out_specs=pl.BlockSpec((OUT_COLS, W2), lambda i: (0, tail_blk)),
        out_shape=jax.ShapeDtypeStruct((OUT_COLS, N_NODES), jnp.float32),
        input_output_aliases={4: 0},
    )(t_t3, x_t3, y_t3, z_t3, feats_t)
    return (attrs_t.T, feats_t.T, spin)
